# Initial kernel scaffold; baseline (speedup 1.0000x reference)
import jax
import jax.numpy as jnp
from jax.experimental import pallas as pl


def kernel(source_feats, target_feats):
    raise NotImplementedError("write your pallas kernel here")


# speedup vs baseline: 1.9098x; 1.9098x over previous
import functools

import jax
import jax.numpy as jnp
from jax import lax
from jax.experimental import pallas as pl
from jax.experimental.pallas import tpu as pltpu
from jax.experimental.pallas import tpu_sc as plsc

K_NN = 4
Q = 2048
T = 32768
D = 1024
TB = 512
NT = T // TB

NEG = float("-inf")
IBIG = 2**30


def _topk_body(src_ref, tgt_ref, idx_out_ref, srcn_ref, vals_ref, idx_ref):
    t = pl.program_id(0)

    @pl.when(t == 0)
    def _init():
        s = src_ref[...]
        n = jnp.sqrt(jnp.sum(s * s, axis=1, keepdims=True)) + 1e-8
        srcn_ref[...] = s / n
        vals_ref[...] = jnp.full((Q, K_NN), NEG, jnp.float32)
        idx_ref[...] = -(jax.lax.broadcasted_iota(jnp.int32, (Q, K_NN), 1) + 1)

    tgt = tgt_ref[...]
    tn = jnp.sqrt(jnp.sum(tgt * tgt, axis=1, keepdims=True)) + 1e-8
    tgt_n = tgt / tn
    s_blk = lax.dot_general(
        srcn_ref[...], tgt_n,
        dimension_numbers=(((1,), (1,)), ((), ())),
        preferred_element_type=jnp.float32,
    )

    cols = jax.lax.broadcasted_iota(jnp.int32, (Q, TB), 1)
    base = t * TB
    blk_vals = []
    blk_idx = []
    for _ in range(K_NN):
        m = jnp.max(s_blk, axis=1)
        am = jnp.min(jnp.where(s_blk == m[:, None], cols, IBIG), axis=1)
        s_blk = jnp.where(cols == am[:, None], NEG, s_blk)
        blk_vals.append(m)
        blk_idx.append(base + am)

    vals8 = jnp.concatenate(
        [vals_ref[...], jnp.stack(blk_vals, axis=1)], axis=1)
    idx8 = jnp.concatenate(
        [idx_ref[...], jnp.stack(blk_idx, axis=1)], axis=1)

    new_vals = []
    new_idx = []
    for _ in range(K_NN):
        m = jnp.max(vals8, axis=1)
        sel = jnp.min(jnp.where(vals8 == m[:, None], idx8, IBIG), axis=1)
        hit = (vals8 == m[:, None]) & (idx8 == sel[:, None])
        vals8 = jnp.where(hit, NEG, vals8)
        new_vals.append(m)
        new_idx.append(sel)

    vals_ref[...] = jnp.stack(new_vals, axis=1)
    idx_ref[...] = jnp.stack(new_idx, axis=1)
    idx_out_ref[...] = idx_ref[...]


def _matmul_topk(source_feats, target_feats):
    return pl.pallas_call(
        _topk_body,
        grid=(NT,),
        in_specs=[
            pl.BlockSpec((Q, D), lambda t: (0, 0)),
            pl.BlockSpec((TB, D), lambda t: (t, 0)),
        ],
        out_specs=pl.BlockSpec((Q, K_NN), lambda t: (0, 0)),
        out_shape=jax.ShapeDtypeStruct((Q, K_NN), jnp.int32),
        scratch_shapes=[
            pltpu.VMEM((Q, D), jnp.float32),
            pltpu.VMEM((Q, K_NN), jnp.float32),
            pltpu.VMEM((Q, K_NN), jnp.int32),
        ],
    )(source_feats, target_feats)


NW = 32
QPW = Q // NW
NCHUNK = 4
QPC = QPW // NCHUNK
RPC = QPC * K_NN
DV = 16


def _gather_mean_body(idx_hbm, tgt_hbm, out_hbm, idx_v, rows_v, acc_v, sem):
    wid = lax.axis_index("s") * 2 + lax.axis_index("c")
    pltpu.sync_copy(idx_hbm.at[wid], idx_v)

    def chunk(g, carry):
        pltpu.async_copy(tgt_hbm.at[idx_v.at[g]], rows_v, sem).wait()

        def per_q(q, c2):
            r = q * K_NN

            def per_d(db, c3):
                s = pl.ds(db * DV, DV)
                v = (rows_v[r, s] + rows_v[r + 1, s]
                     + rows_v[r + 2, s] + rows_v[r + 3, s]) * 0.25
                acc_v[q, s] = v
                return c3

            return lax.fori_loop(0, D // DV, per_d, c2)

        lax.fori_loop(0, QPC, per_q, carry)
        pltpu.sync_copy(acc_v, out_hbm.at[pl.ds(wid * QPW + g * QPC, QPC)])
        return carry

    lax.fori_loop(0, NCHUNK, chunk, 0)


def _gather_mean(idx_resh, target_feats):
    mesh = plsc.VectorSubcoreMesh(core_axis_name="c", subcore_axis_name="s")
    f = functools.partial(
        pl.kernel,
        out_type=jax.ShapeDtypeStruct((Q, D), jnp.float32),
        mesh=mesh,
        scratch_types=[
            pltpu.VMEM((NCHUNK, RPC), jnp.int32),
            pltpu.VMEM((RPC, D), jnp.float32),
            pltpu.VMEM((QPC, D), jnp.float32),
            pltpu.SemaphoreType.DMA,
        ],
    )(_gather_mean_body)
    return f(idx_resh, target_feats)


def kernel(source_feats, target_feats):
    top_idx = _matmul_topk(source_feats, target_feats)
    idx_resh = top_idx.reshape(NW, NCHUNK, RPC)
    return _gather_mean(idx_resh, target_feats)


# speedup vs baseline: 2.4499x; 1.2828x over previous
import functools

import jax
import jax.numpy as jnp
from jax import lax
from jax.experimental import pallas as pl
from jax.experimental.pallas import tpu as pltpu
from jax.experimental.pallas import tpu_sc as plsc

K_NN = 4
Q = 2048
T = 32768
D = 1024
TB = 512
NT = T // TB
QS = Q // 128

NEG = float("-inf")
IBIG = 2**30


def _matmul_filter_body(src_ref, tgt_ref, sims_ref, seg_ref,
                        srcn_ref, smax_ref):
    t = pl.program_id(0)

    @pl.when(t == 0)
    def _init():
        s = src_ref[...]
        n = jnp.sqrt(jnp.sum(s * s, axis=1, keepdims=True)) + 1e-8
        srcn_ref[...] = s / n

    tgt = tgt_ref[...]
    tn = jnp.sqrt(jnp.sum(tgt * tgt, axis=1, keepdims=True)) + 1e-8
    tgt_n = tgt / tn
    s_blk = lax.dot_general(
        srcn_ref[...], tgt_n,
        dimension_numbers=(((1,), (1,)), ((), ())),
        preferred_element_type=jnp.float32,
    )
    sims_ref[...] = s_blk
    m = jnp.max(s_blk, axis=1)
    smax_ref[pl.ds(t, 1)] = m.reshape(1, QS, 128)

    @pl.when(t == NT - 1)
    def _extract():
        smax = smax_ref[...]
        segs = jax.lax.broadcasted_iota(jnp.int32, (NT, QS, 128), 0)
        sel_list = []
        for _ in range(K_NN):
            mm = jnp.max(smax, axis=0)
            sel = jnp.min(
                jnp.where(smax == mm[None], segs, IBIG), axis=0)
            hit = (smax == mm[None]) & (segs == sel[None])
            smax = jnp.where(hit, NEG, smax)
            sel_list.append(sel)
        seg_ref[...] = jnp.stack(sel_list, axis=0)


def _matmul_filter(source_feats, target_feats):
    return pl.pallas_call(
        _matmul_filter_body,
        grid=(NT,),
        in_specs=[
            pl.BlockSpec((Q, D), lambda t: (0, 0)),
            pl.BlockSpec((TB, D), lambda t: (t, 0)),
        ],
        out_specs=[
            pl.BlockSpec((Q, TB), lambda t: (0, t)),
            pl.BlockSpec((K_NN, QS, 128), lambda t: (0, 0, 0)),
        ],
        out_shape=[
            jax.ShapeDtypeStruct((Q, T), jnp.float32),
            jax.ShapeDtypeStruct((K_NN, QS, 128), jnp.int32),
        ],
        scratch_shapes=[
            pltpu.VMEM((Q, D), jnp.float32),
            pltpu.VMEM((NT, QS, 128), jnp.float32),
        ],
    )(source_feats, target_feats)


NW = 32
DV = 16


def _gather_rows_body(nb, rows_per_batch, row_w, idx_hbm, src_hbm, out_hbm,
                      idx_v, buf_v, sem):
    wid = lax.axis_index("s") * 2 + lax.axis_index("c")
    pltpu.sync_copy(idx_hbm.at[wid], idx_v)

    def batch(h, carry):
        pltpu.async_copy(src_hbm.at[idx_v.at[h]], buf_v, sem).wait()
        base = (wid * nb + h) * rows_per_batch
        pltpu.sync_copy(buf_v, out_hbm.at[pl.ds(base, rows_per_batch)])
        return carry

    lax.fori_loop(0, nb, batch, 0)


def _gather_rows(idx, src, nb, rows_per_batch, row_w):
    nrows = NW * nb * rows_per_batch
    mesh = plsc.VectorSubcoreMesh(core_axis_name="c", subcore_axis_name="s")
    body = functools.partial(_gather_rows_body, nb, rows_per_batch, row_w)
    f = functools.partial(
        pl.kernel,
        out_type=jax.ShapeDtypeStruct((nrows, row_w), jnp.float32),
        mesh=mesh,
        scratch_types=[
            pltpu.VMEM((nb, rows_per_batch), jnp.int32),
            pltpu.VMEM((rows_per_batch, row_w), jnp.float32),
            pltpu.SemaphoreType.DMA,
        ],
    )(body)
    return f(idx, src)


QPW = Q // NW
NCHUNK = 4
QPC = QPW // NCHUNK
RPC = QPC * K_NN


def _gather_mean_body(idx_hbm, tgt_hbm, out_hbm, idx_v, rows_v, acc_v, sem):
    wid = lax.axis_index("s") * 2 + lax.axis_index("c")
    pltpu.sync_copy(idx_hbm.at[wid], idx_v)

    def chunk(g, carry):
        pltpu.async_copy(tgt_hbm.at[idx_v.at[g]], rows_v, sem).wait()

        def per_q(q, c2):
            r = q * K_NN

            def per_d(db, c3):
                s = pl.ds(db * DV, DV)
                v = (rows_v[r, s] + rows_v[r + 1, s]
                     + rows_v[r + 2, s] + rows_v[r + 3, s]) * 0.25
                acc_v[q, s] = v
                return c3

            return lax.fori_loop(0, D // DV, per_d, c2)

        lax.fori_loop(0, QPC, per_q, carry)
        pltpu.sync_copy(acc_v, out_hbm.at[pl.ds(wid * QPW + g * QPC, QPC)])
        return carry

    lax.fori_loop(0, NCHUNK, chunk, 0)


def _gather_mean(idx_resh, target_feats):
    mesh = plsc.VectorSubcoreMesh(core_axis_name="c", subcore_axis_name="s")
    f = functools.partial(
        pl.kernel,
        out_type=jax.ShapeDtypeStruct((Q, D), jnp.float32),
        mesh=mesh,
        scratch_types=[
            pltpu.VMEM((NCHUNK, RPC), jnp.int32),
            pltpu.VMEM((RPC, D), jnp.float32),
            pltpu.VMEM((QPC, D), jnp.float32),
            pltpu.SemaphoreType.DMA,
        ],
    )(_gather_mean_body)
    return f(idx_resh, target_feats)


NCAND = K_NN * TB


def _exact_topk_body(cand_ref, segt_ref, idx_out_ref):
    cand = cand_ref[...]
    segt = segt_ref[...]
    cols = jax.lax.broadcasted_iota(jnp.int32, (Q, NCAND), 1)
    j = cols // TB
    off = cols - j * TB
    gidx = off
    for jj in range(K_NN):
        gidx = gidx + jnp.where(j == jj, segt[:, jj][:, None] * TB, 0)

    out = []
    for _ in range(K_NN):
        m = jnp.max(cand, axis=1)
        sel = jnp.min(jnp.where(cand == m[:, None], gidx, IBIG), axis=1)
        hit = (cand == m[:, None]) & (gidx == sel[:, None])
        cand = jnp.where(hit, NEG, cand)
        out.append(sel)
    idx_out_ref[...] = jnp.stack(out, axis=1)


def _exact_topk(cand, seg_t):
    return pl.pallas_call(
        _exact_topk_body,
        in_specs=[
            pl.BlockSpec((Q, NCAND), lambda: (0, 0)),
            pl.BlockSpec((Q, K_NN), lambda: (0, 0)),
        ],
        out_specs=pl.BlockSpec((Q, K_NN), lambda: (0, 0)),
        out_shape=jax.ShapeDtypeStruct((Q, K_NN), jnp.int32),
    )(cand, seg_t)


def kernel(source_feats, target_feats):
    sims, seg_ids = _matmul_filter(source_feats, target_feats)
    seg_t = seg_ids.reshape(K_NN, Q).T
    qv = jnp.arange(Q, dtype=jnp.int32)[:, None]
    cand_rows = (qv * NT + seg_t).reshape(NW, 2, 128)
    sims_rows = sims.reshape(Q * NT, TB)
    cand = _gather_rows(cand_rows, sims_rows, 2, 128, TB)
    top_idx = _exact_topk(cand.reshape(Q, NCAND), seg_t)
    idx_resh = top_idx.reshape(NW, NCHUNK, RPC)
    return _gather_mean(idx_resh, target_feats)
